# baseline (speedup 1.0000x reference)
import dataclasses
import functools

import jax
import jax.numpy as jnp
from jax import lax
from jax.experimental import pallas as pl
from jax.experimental.pallas import tpu as pltpu
from jax.experimental.pallas import tpu_sc as plsc

N = 10000
E = 320000
H = 128
G = 64
C = 10
EPS = 1e-5

NC = 2
NS = 16
NW = NC * NS
K = 128
CPT = 80
E_PAD = NW * CPT * K
NP = 10240
RPT = NP // NS
ZB = 64
DUMP = NP - 1

_mesh = plsc.VectorSubcoreMesh(
    core_axis_name="c", subcore_axis_name="s", num_cores=NC, num_subcores=NS
)
_sc_no_layout = dataclasses.replace(
    pltpu.CompilerParams(), needs_layout_passes=False
)


@functools.partial(
    pl.kernel,
    out_type=jax.ShapeDtypeStruct((NW, NP), jnp.float32),
    mesh=_mesh,
    scratch_types=[
        pltpu.VMEM((K,), jnp.int32),
        pltpu.VMEM((NP,), jnp.float32),
    ],
    compiler_params=_sc_no_layout,
)
def _sc_degree(e_hbm, out, colbuf, hist_v):
    cid = lax.axis_index("c")
    sid = lax.axis_index("s")
    wid = cid * NS + sid

    @pl.loop(0, NP, step=16)
    def _(r):
        hist_v[pl.ds(r, 16)] = jnp.zeros((16,), jnp.float32)

    ones16 = jnp.ones((16,), jnp.float32)

    @pl.loop(0, CPT)
    def _(j):
        pltpu.sync_copy(e_hbm.at[wid, j, 1], colbuf)

        @pl.loop(0, K, step=16)
        def _(q):
            idx = colbuf[pl.ds(q, 16)]
            plsc.addupdate_scatter(hist_v, [idx], ones16)

    pltpu.sync_copy(hist_v, out.at[wid])


@functools.partial(
    pl.kernel,
    out_type=jax.ShapeDtypeStruct((2 * NP, H), jnp.float32),
    mesh=_mesh,
    scratch_types=[
        pltpu.VMEM((2, K), jnp.int32),
        pltpu.VMEM((2, K), jnp.int32),
        pltpu.VMEM((K, H), jnp.float32),
        pltpu.VMEM((K, H), jnp.float32),
        pltpu.VMEM_SHARED((NP, H), jnp.float32),
        pltpu.SemaphoreType.DMA,
        pltpu.SemaphoreType.DMA,
        pltpu.SemaphoreType.DMA,
    ],
)
def _sc_aggregate(xs_hbm, e_hbm, zero_hbm, out, idx0, idx1, buf0, buf1,
                  acc, semi0, semi1, semw):
    cid = lax.axis_index("c")
    sid = lax.axis_index("s")
    wid = cid * NS + sid

    pltpu.sync_copy(zero_hbm, buf0)

    @pl.loop(0, RPT, step=K)
    def _(r):
        pltpu.async_copy(buf0, acc.at[pl.ds(sid * RPT + r, K)], semw)

    @pl.loop(0, RPT, step=K)
    def _(r):
        pltpu.make_async_copy(buf0, acc.at[pl.ds(sid * RPT + r, K)], semw).wait()

    plsc.subcore_barrier()

    @pl.loop(0, CPT)
    def _(j):
        pltpu.sync_copy(e_hbm.at[wid, j], idx0)
        pltpu.async_copy(xs_hbm.at[idx0.at[0]], buf0, semi0).wait()
        pltpu.sync_copy(buf0, acc.at[idx0.at[1]], add=True)

    plsc.subcore_barrier()

    nq = RPT // K
    for q in range(nq):
        b = buf0 if q % 2 == 0 else buf1
        if q >= 2:
            bp = buf0 if (q - 2) % 2 == 0 else buf1
            pltpu.make_async_copy(
                bp, out.at[pl.ds(cid * NP + sid * RPT + (q - 2) * K, K)], semw
            ).wait()
        pltpu.sync_copy(acc.at[pl.ds(sid * RPT + q * K, K)], b)
        pltpu.async_copy(b, out.at[pl.ds(cid * NP + sid * RPT + q * K, K)], semw)

    for q in range(max(nq - 2, 0), nq):
        b = buf0 if q % 2 == 0 else buf1
        pltpu.make_async_copy(
            b, out.at[pl.ds(cid * NP + sid * RPT + q * K, K)], semw
        ).wait()


_BLK = 1024
_NBLK = NP // _BLK


def _tc_dinv(degs):
    def body(d_ref, o_ref):
        s = jnp.sum(d_ref[...], axis=0, keepdims=True) + 1.0
        o_ref[...] = jnp.transpose(lax.rsqrt(s))

    return pl.pallas_call(
        body,
        out_shape=jax.ShapeDtypeStruct((NP, 1), jnp.float32),
    )(degs)


def _tc_matmul_scale(h, W, dinv):
    def body(h_ref, w_ref, d_ref, o_ref):
        o_ref[...] = d_ref[...] * jnp.dot(
            h_ref[...], w_ref[...], preferred_element_type=jnp.float32
        )

    return pl.pallas_call(
        body,
        grid=(_NBLK,),
        in_specs=[
            pl.BlockSpec((_BLK, H), lambda i: (i, 0)),
            pl.BlockSpec((H, H), lambda i: (0, 0)),
            pl.BlockSpec((_BLK, 1), lambda i: (i, 0)),
        ],
        out_specs=pl.BlockSpec((_BLK, H), lambda i: (i, 0)),
        out_shape=jax.ShapeDtypeStruct((NP, H), jnp.float32),
    )(h, W, dinv)


def _tc_combine(p, xs, dinv, b, g, be):
    def body(p0_ref, p1_ref, xs_ref, d_ref, b_ref, g_ref, be_ref, o_ref):
        gg = g_ref[...] * lax.rsqrt(jnp.float32(1.0 + EPS))
        agg = d_ref[...] * (p0_ref[...] + p1_ref[...] + xs_ref[...]) + b_ref[...]
        o_ref[...] = jnp.maximum(gg * agg + be_ref[...], 0.0)

    return pl.pallas_call(
        body,
        grid=(_NBLK,),
        in_specs=[
            pl.BlockSpec((_BLK, H), lambda i: (i, 0)),
            pl.BlockSpec((_BLK, H), lambda i: (_NBLK + i, 0)),
            pl.BlockSpec((_BLK, H), lambda i: (i, 0)),
            pl.BlockSpec((_BLK, 1), lambda i: (i, 0)),
            pl.BlockSpec((1, H), lambda i: (0, 0)),
            pl.BlockSpec((1, H), lambda i: (0, 0)),
            pl.BlockSpec((1, H), lambda i: (0, 0)),
        ],
        out_specs=pl.BlockSpec((_BLK, H), lambda i: (i, 0)),
        out_shape=jax.ShapeDtypeStruct((NP, H), jnp.float32),
    )(p, p, xs, dinv, b.reshape(1, H), g.reshape(1, H), be.reshape(1, H))


def _tc_pool_head(h, batch2d, lw1, lb1, g4, be4, lw2, lb2):
    def body(h_ref, b_ref, lw1_ref, lb1_ref, g4_ref, be4_ref, lw2_ref, lb2_ref,
             o_ref, sums, cnt):
        i = pl.program_id(0)

        @pl.when(i == 0)
        def _():
            sums[...] = jnp.zeros_like(sums)
            cnt[...] = jnp.zeros_like(cnt)

        seg = b_ref[...]
        onehot = (seg == lax.broadcasted_iota(jnp.int32, (_BLK, G), 1)).astype(
            jnp.float32
        )
        sums[...] += lax.dot_general(
            onehot, h_ref[...], (((0,), (0,)), ((), ())),
            preferred_element_type=jnp.float32,
        )
        cnt[...] += jnp.sum(onehot, axis=0, keepdims=True)

        @pl.when(i == _NBLK - 1)
        def _():
            p = sums[...] / jnp.maximum(cnt[...], 1.0).T
            gg = g4_ref[...] * lax.rsqrt(jnp.float32(1.0 + EPS))
            q = jnp.dot(p, lw1_ref[...], preferred_element_type=jnp.float32)
            q = jnp.maximum(gg * (q + lb1_ref[...]) + be4_ref[...], 0.0)
            o_ref[...] = (
                jnp.dot(q, lw2_ref[...], preferred_element_type=jnp.float32)
                + lb2_ref[...]
            )

    return pl.pallas_call(
        body,
        grid=(_NBLK,),
        in_specs=[
            pl.BlockSpec((_BLK, H), lambda i: (i, 0)),
            pl.BlockSpec((_BLK, 1), lambda i: (i, 0)),
            pl.BlockSpec((H, H), lambda i: (0, 0)),
            pl.BlockSpec((1, H), lambda i: (0, 0)),
            pl.BlockSpec((1, H), lambda i: (0, 0)),
            pl.BlockSpec((1, H), lambda i: (0, 0)),
            pl.BlockSpec((H, C), lambda i: (0, 0)),
            pl.BlockSpec((1, C), lambda i: (0, 0)),
        ],
        out_specs=pl.BlockSpec((G, C), lambda i: (0, 0)),
        out_shape=jax.ShapeDtypeStruct((G, C), jnp.float32),
        scratch_shapes=[
            pltpu.VMEM((G, H), jnp.float32),
            pltpu.VMEM((1, G), jnp.float32),
        ],
    )(h, batch2d, lw1, lb1.reshape(1, H), g4.reshape(1, H), be4.reshape(1, H),
      lw2, lb2.reshape(1, C))


def kernel(x, edge_index, batch, W1, b1, g1, be1, W2, b2, g2, be2,
           W3, b3, g3, be3, lw1, lb1, g4, be4, lw2, lb2):
    row = edge_index[0]
    col = edge_index[1]
    pad = E_PAD - E
    row3 = jnp.concatenate([row, jnp.zeros((pad,), jnp.int32)]).reshape(NW, CPT, K)
    col3 = jnp.concatenate([col, jnp.full((pad,), DUMP, jnp.int32)]).reshape(NW, CPT, K)
    e3 = jnp.stack([row3, col3], axis=2)
    x_p = jnp.concatenate([x, jnp.zeros((NP - N, x.shape[1]), x.dtype)])
    batch2d = jnp.concatenate([batch, jnp.full((NP - N,), G, batch.dtype)])
    batch2d = batch2d.reshape(NP, 1)
    zrows = jnp.zeros((K, H), jnp.float32)

    degs = _sc_degree(e3)
    dinv = _tc_dinv(degs)

    h = x_p
    for W, b, g, be in ((W1, b1, g1, be1), (W2, b2, g2, be2), (W3, b3, g3, be3)):
        xs = _tc_matmul_scale(h, W, dinv)
        p = _sc_aggregate(xs, e3, zrows)
        h = _tc_combine(p, xs, dinv, b, g, be)

    return _tc_pool_head(h, batch2d, lw1, lb1, g4, be4, lw2, lb2)


# speedup vs baseline: 1.2225x; 1.2225x over previous
import dataclasses
import functools

import jax
import jax.numpy as jnp
from jax import lax
from jax.experimental import pallas as pl
from jax.experimental.pallas import tpu as pltpu
from jax.experimental.pallas import tpu_sc as plsc

N = 10000
E = 320000
H = 128
G = 64
C = 10
EPS = 1e-5

NC = 2
NS = 16
NW = NC * NS
K = 128
CPT = 80
E_PAD = NW * CPT * K
NP = 10240
RPT = NP // NS
ZB = 64
DUMP = NP - 1

_mesh = plsc.VectorSubcoreMesh(
    core_axis_name="c", subcore_axis_name="s", num_cores=NC, num_subcores=NS
)
_sc_no_layout = dataclasses.replace(
    pltpu.CompilerParams(), needs_layout_passes=False
)


@functools.partial(
    pl.kernel,
    out_type=jax.ShapeDtypeStruct((NW, NP), jnp.float32),
    mesh=_mesh,
    scratch_types=[
        pltpu.VMEM((K,), jnp.int32),
        pltpu.VMEM((NP,), jnp.float32),
    ],
    compiler_params=_sc_no_layout,
)
def _sc_degree(col_hbm, out, colbuf, hist_v):
    cid = lax.axis_index("c")
    sid = lax.axis_index("s")
    wid = cid * NS + sid

    @pl.loop(0, NP, step=16)
    def _(r):
        hist_v[pl.ds(r, 16)] = jnp.zeros((16,), jnp.float32)

    ones16 = jnp.ones((16,), jnp.float32)

    @pl.loop(0, CPT)
    def _(j):
        pltpu.sync_copy(col_hbm.at[wid, j], colbuf)

        @pl.loop(0, K, step=16)
        def _(q):
            idx = colbuf[pl.ds(q, 16)]
            plsc.addupdate_scatter(hist_v, [idx], ones16)

    pltpu.sync_copy(hist_v, out.at[wid])


@functools.partial(
    pl.kernel,
    out_type=jax.ShapeDtypeStruct((2 * NP, H), jnp.float32),
    mesh=_mesh,
    scratch_types=[
        pltpu.VMEM((K,), jnp.int32),
        pltpu.VMEM((K,), jnp.int32),
        pltpu.VMEM((K, H), jnp.float32),
        pltpu.VMEM((K, H), jnp.float32),
        pltpu.VMEM_SHARED((NP, H), jnp.float32),
        pltpu.SemaphoreType.DMA,
        pltpu.SemaphoreType.DMA,
        pltpu.SemaphoreType.DMA,
    ],
)
def _sc_aggregate(xs_hbm, row_hbm, col_hbm, zero_hbm, out, rowbuf, colbuf,
                  buf0, buf1, acc, semi0, semi1, semw):
    cid = lax.axis_index("c")
    sid = lax.axis_index("s")
    wid = cid * NS + sid

    pltpu.sync_copy(zero_hbm, buf0)

    @pl.loop(0, RPT, step=K)
    def _(r):
        pltpu.async_copy(buf0, acc.at[pl.ds(sid * RPT + r, K)], semw)

    @pl.loop(0, RPT, step=K)
    def _(r):
        pltpu.make_async_copy(buf0, acc.at[pl.ds(sid * RPT + r, K)], semw).wait()

    plsc.subcore_barrier()

    @pl.loop(0, CPT)
    def _(j):
        pltpu.sync_copy(row_hbm.at[wid, j], rowbuf)
        pltpu.sync_copy(col_hbm.at[wid, j], colbuf)
        pltpu.async_copy(xs_hbm.at[rowbuf], buf0, semi0).wait()
        pltpu.sync_copy(buf0, acc.at[colbuf], add=True)

    plsc.subcore_barrier()

    nq = RPT // K
    for q in range(nq):
        b = buf0 if q % 2 == 0 else buf1
        if q >= 2:
            bp = buf0 if (q - 2) % 2 == 0 else buf1
            pltpu.make_async_copy(
                bp, out.at[pl.ds(cid * NP + sid * RPT + (q - 2) * K, K)], semw
            ).wait()
        pltpu.sync_copy(acc.at[pl.ds(sid * RPT + q * K, K)], b)
        pltpu.async_copy(b, out.at[pl.ds(cid * NP + sid * RPT + q * K, K)], semw)

    for q in range(max(nq - 2, 0), nq):
        b = buf0 if q % 2 == 0 else buf1
        pltpu.make_async_copy(
            b, out.at[pl.ds(cid * NP + sid * RPT + q * K, K)], semw
        ).wait()


_BLK = 1024
_NBLK = NP // _BLK


def _tc_dinv(degs):
    def body(d_ref, o_ref):
        s = jnp.sum(d_ref[...], axis=0, keepdims=True) + 1.0
        o_ref[...] = jnp.transpose(lax.rsqrt(s))

    return pl.pallas_call(
        body,
        out_shape=jax.ShapeDtypeStruct((NP, 1), jnp.float32),
    )(degs)


def _tc_matmul_scale(h, W, dinv):
    def body(h_ref, w_ref, d_ref, o_ref):
        o_ref[...] = d_ref[...] * jnp.dot(
            h_ref[...], w_ref[...], preferred_element_type=jnp.float32
        )

    return pl.pallas_call(
        body,
        grid=(_NBLK,),
        in_specs=[
            pl.BlockSpec((_BLK, H), lambda i: (i, 0)),
            pl.BlockSpec((H, H), lambda i: (0, 0)),
            pl.BlockSpec((_BLK, 1), lambda i: (i, 0)),
        ],
        out_specs=pl.BlockSpec((_BLK, H), lambda i: (i, 0)),
        out_shape=jax.ShapeDtypeStruct((NP, H), jnp.float32),
    )(h, W, dinv)


def _tc_combine(p, xs, dinv, b, g, be):
    def body(p0_ref, p1_ref, xs_ref, d_ref, b_ref, g_ref, be_ref, o_ref):
        gg = g_ref[...] * lax.rsqrt(jnp.float32(1.0 + EPS))
        agg = d_ref[...] * (p0_ref[...] + p1_ref[...] + xs_ref[...]) + b_ref[...]
        o_ref[...] = jnp.maximum(gg * agg + be_ref[...], 0.0)

    return pl.pallas_call(
        body,
        grid=(_NBLK,),
        in_specs=[
            pl.BlockSpec((_BLK, H), lambda i: (i, 0)),
            pl.BlockSpec((_BLK, H), lambda i: (_NBLK + i, 0)),
            pl.BlockSpec((_BLK, H), lambda i: (i, 0)),
            pl.BlockSpec((_BLK, 1), lambda i: (i, 0)),
            pl.BlockSpec((1, H), lambda i: (0, 0)),
            pl.BlockSpec((1, H), lambda i: (0, 0)),
            pl.BlockSpec((1, H), lambda i: (0, 0)),
        ],
        out_specs=pl.BlockSpec((_BLK, H), lambda i: (i, 0)),
        out_shape=jax.ShapeDtypeStruct((NP, H), jnp.float32),
    )(p, p, xs, dinv, b.reshape(1, H), g.reshape(1, H), be.reshape(1, H))


def _tc_pool_head(h, batch2d, lw1, lb1, g4, be4, lw2, lb2):
    def body(h_ref, b_ref, lw1_ref, lb1_ref, g4_ref, be4_ref, lw2_ref, lb2_ref,
             o_ref, sums, cnt):
        i = pl.program_id(0)

        @pl.when(i == 0)
        def _():
            sums[...] = jnp.zeros_like(sums)
            cnt[...] = jnp.zeros_like(cnt)

        seg = b_ref[...]
        onehot = (seg == lax.broadcasted_iota(jnp.int32, (_BLK, G), 1)).astype(
            jnp.float32
        )
        sums[...] += lax.dot_general(
            onehot, h_ref[...], (((0,), (0,)), ((), ())),
            preferred_element_type=jnp.float32,
        )
        cnt[...] += jnp.sum(onehot, axis=0, keepdims=True)

        @pl.when(i == _NBLK - 1)
        def _():
            p = sums[...] / jnp.maximum(cnt[...], 1.0).T
            gg = g4_ref[...] * lax.rsqrt(jnp.float32(1.0 + EPS))
            q = jnp.dot(p, lw1_ref[...], preferred_element_type=jnp.float32)
            q = jnp.maximum(gg * (q + lb1_ref[...]) + be4_ref[...], 0.0)
            o_ref[...] = (
                jnp.dot(q, lw2_ref[...], preferred_element_type=jnp.float32)
                + lb2_ref[...]
            )

    return pl.pallas_call(
        body,
        grid=(_NBLK,),
        in_specs=[
            pl.BlockSpec((_BLK, H), lambda i: (i, 0)),
            pl.BlockSpec((_BLK, 1), lambda i: (i, 0)),
            pl.BlockSpec((H, H), lambda i: (0, 0)),
            pl.BlockSpec((1, H), lambda i: (0, 0)),
            pl.BlockSpec((1, H), lambda i: (0, 0)),
            pl.BlockSpec((1, H), lambda i: (0, 0)),
            pl.BlockSpec((H, C), lambda i: (0, 0)),
            pl.BlockSpec((1, C), lambda i: (0, 0)),
        ],
        out_specs=pl.BlockSpec((G, C), lambda i: (0, 0)),
        out_shape=jax.ShapeDtypeStruct((G, C), jnp.float32),
        scratch_shapes=[
            pltpu.VMEM((G, H), jnp.float32),
            pltpu.VMEM((1, G), jnp.float32),
        ],
    )(h, batch2d, lw1, lb1.reshape(1, H), g4.reshape(1, H), be4.reshape(1, H),
      lw2, lb2.reshape(1, C))


def kernel(x, edge_index, batch, W1, b1, g1, be1, W2, b2, g2, be2,
           W3, b3, g3, be3, lw1, lb1, g4, be4, lw2, lb2):
    row = edge_index[0]
    col = edge_index[1]
    pad = E_PAD - E
    row3 = jnp.concatenate([row, jnp.zeros((pad,), jnp.int32)]).reshape(NW, CPT, K)
    col3 = jnp.concatenate([col, jnp.full((pad,), DUMP, jnp.int32)]).reshape(NW, CPT, K)
    x_p = jnp.concatenate([x, jnp.zeros((NP - N, x.shape[1]), x.dtype)])
    batch2d = jnp.concatenate([batch, jnp.full((NP - N,), G, batch.dtype)])
    batch2d = batch2d.reshape(NP, 1)
    zrows = jnp.zeros((K, H), jnp.float32)

    degs = _sc_degree(col3)
    dinv = _tc_dinv(degs)

    h = x_p
    for W, b, g, be in ((W1, b1, g1, be1), (W2, b2, g2, be2), (W3, b3, g3, be3)):
        xs = _tc_matmul_scale(h, W, dinv)
        p = _sc_aggregate(xs, row3, col3, zrows)
        h = _tc_combine(p, xs, dinv, b, g, be)

    return _tc_pool_head(h, batch2d, lw1, lb1, g4, be4, lw2, lb2)


# speedup vs baseline: 1.6478x; 1.3478x over previous
import dataclasses
import functools

import jax
import jax.numpy as jnp
from jax import lax
from jax.experimental import pallas as pl
from jax.experimental.pallas import tpu as pltpu
from jax.experimental.pallas import tpu_sc as plsc

N = 10000
E = 320000
H = 128
G = 64
C = 10
EPS = 1e-5

NC = 2
NS = 16
NW = NC * NS
K = 128
CPT = 79
E_PAD = NW * CPT * K
NP = 10240
RPT = NP // NS
ZB = 64
DUMP = NP - 1

_mesh = plsc.VectorSubcoreMesh(
    core_axis_name="c", subcore_axis_name="s", num_cores=NC, num_subcores=NS
)
_sc_no_layout = dataclasses.replace(
    pltpu.CompilerParams(), needs_layout_passes=False
)


@functools.partial(
    pl.kernel,
    out_type=jax.ShapeDtypeStruct((NW, NP), jnp.float32),
    mesh=_mesh,
    scratch_types=[
        pltpu.VMEM((K,), jnp.int32),
        pltpu.VMEM((NP,), jnp.float32),
    ],
    compiler_params=_sc_no_layout,
)
def _sc_degree(col_hbm, out, colbuf, hist_v):
    cid = lax.axis_index("c")
    sid = lax.axis_index("s")
    wid = cid * NS + sid

    @pl.loop(0, NP, step=16)
    def _(r):
        hist_v[pl.ds(r, 16)] = jnp.zeros((16,), jnp.float32)

    ones16 = jnp.ones((16,), jnp.float32)

    @pl.loop(0, CPT)
    def _(j):
        pltpu.sync_copy(col_hbm.at[wid, j], colbuf)

        @pl.loop(0, K, step=16)
        def _(q):
            idx = colbuf[pl.ds(q, 16)]
            plsc.addupdate_scatter(hist_v, [idx], ones16)

    pltpu.sync_copy(hist_v, out.at[wid])


@functools.partial(
    pl.kernel,
    out_type=jax.ShapeDtypeStruct((2 * NP, H), jnp.float32),
    mesh=_mesh,
    scratch_types=[
        pltpu.VMEM((K,), jnp.int32),
        pltpu.VMEM((K,), jnp.int32),
        pltpu.VMEM((K, H), jnp.float32),
        pltpu.VMEM((ZB, H), jnp.float32),
        pltpu.VMEM_SHARED((NP, H), jnp.float32),
        pltpu.SemaphoreType.DMA,
    ],
)
def _sc_aggregate(xs_hbm, row_hbm, col_hbm, zero_hbm, out, rowbuf, colbuf,
                  buf, z_v, acc, sem):
    cid = lax.axis_index("c")
    sid = lax.axis_index("s")
    wid = cid * NS + sid

    pltpu.sync_copy(zero_hbm, z_v)

    @pl.loop(0, RPT, step=ZB)
    def _(r):
        pltpu.sync_copy(z_v, acc.at[pl.ds(sid * RPT + r, ZB)])

    plsc.subcore_barrier()

    @pl.loop(0, CPT)
    def _(j):
        pltpu.sync_copy(row_hbm.at[wid, j], rowbuf)
        pltpu.sync_copy(col_hbm.at[wid, j], colbuf)
        pltpu.async_copy(xs_hbm.at[rowbuf], buf, sem).wait()
        pltpu.sync_copy(buf, acc.at[colbuf], add=True)

    plsc.subcore_barrier()

    @pl.loop(0, RPT, step=ZB)
    def _(r):
        pltpu.sync_copy(acc.at[pl.ds(sid * RPT + r, ZB)], z_v)
        pltpu.sync_copy(z_v, out.at[pl.ds(cid * NP + sid * RPT + r, ZB)])


_BLK = 1024
_NBLK = NP // _BLK


def _tc_dinv(degs):
    def body(d_ref, o_ref):
        s = jnp.sum(d_ref[...], axis=0, keepdims=True) + 1.0
        o_ref[...] = jnp.transpose(lax.rsqrt(s))

    return pl.pallas_call(
        body,
        out_shape=jax.ShapeDtypeStruct((NP, 1), jnp.float32),
    )(degs)


def _tc_matmul_scale(h, W, dinv):
    def body(h_ref, w_ref, d_ref, o_ref):
        o_ref[...] = d_ref[...] * jnp.dot(
            h_ref[...], w_ref[...], preferred_element_type=jnp.float32
        )

    return pl.pallas_call(
        body,
        grid=(_NBLK,),
        in_specs=[
            pl.BlockSpec((_BLK, H), lambda i: (i, 0)),
            pl.BlockSpec((H, H), lambda i: (0, 0)),
            pl.BlockSpec((_BLK, 1), lambda i: (i, 0)),
        ],
        out_specs=pl.BlockSpec((_BLK, H), lambda i: (i, 0)),
        out_shape=jax.ShapeDtypeStruct((NP, H), jnp.float32),
    )(h, W, dinv)


def _tc_combine(p, xs, dinv, b, g, be):
    def body(p0_ref, p1_ref, xs_ref, d_ref, b_ref, g_ref, be_ref, o_ref):
        gg = g_ref[...] * lax.rsqrt(jnp.float32(1.0 + EPS))
        agg = d_ref[...] * (p0_ref[...] + p1_ref[...] + xs_ref[...]) + b_ref[...]
        o_ref[...] = jnp.maximum(gg * agg + be_ref[...], 0.0)

    return pl.pallas_call(
        body,
        grid=(_NBLK,),
        in_specs=[
            pl.BlockSpec((_BLK, H), lambda i: (i, 0)),
            pl.BlockSpec((_BLK, H), lambda i: (_NBLK + i, 0)),
            pl.BlockSpec((_BLK, H), lambda i: (i, 0)),
            pl.BlockSpec((_BLK, 1), lambda i: (i, 0)),
            pl.BlockSpec((1, H), lambda i: (0, 0)),
            pl.BlockSpec((1, H), lambda i: (0, 0)),
            pl.BlockSpec((1, H), lambda i: (0, 0)),
        ],
        out_specs=pl.BlockSpec((_BLK, H), lambda i: (i, 0)),
        out_shape=jax.ShapeDtypeStruct((NP, H), jnp.float32),
    )(p, p, xs, dinv, b.reshape(1, H), g.reshape(1, H), be.reshape(1, H))


def _tc_pool_head(h, batch2d, lw1, lb1, g4, be4, lw2, lb2):
    def body(h_ref, b_ref, lw1_ref, lb1_ref, g4_ref, be4_ref, lw2_ref, lb2_ref,
             o_ref, sums, cnt):
        i = pl.program_id(0)

        @pl.when(i == 0)
        def _():
            sums[...] = jnp.zeros_like(sums)
            cnt[...] = jnp.zeros_like(cnt)

        seg = b_ref[...]
        onehot = (seg == lax.broadcasted_iota(jnp.int32, (_BLK, G), 1)).astype(
            jnp.float32
        )
        sums[...] += lax.dot_general(
            onehot, h_ref[...], (((0,), (0,)), ((), ())),
            preferred_element_type=jnp.float32,
        )
        cnt[...] += jnp.sum(onehot, axis=0, keepdims=True)

        @pl.when(i == _NBLK - 1)
        def _():
            p = sums[...] / jnp.maximum(cnt[...], 1.0).T
            gg = g4_ref[...] * lax.rsqrt(jnp.float32(1.0 + EPS))
            q = jnp.dot(p, lw1_ref[...], preferred_element_type=jnp.float32)
            q = jnp.maximum(gg * (q + lb1_ref[...]) + be4_ref[...], 0.0)
            o_ref[...] = (
                jnp.dot(q, lw2_ref[...], preferred_element_type=jnp.float32)
                + lb2_ref[...]
            )

    return pl.pallas_call(
        body,
        grid=(_NBLK,),
        in_specs=[
            pl.BlockSpec((_BLK, H), lambda i: (i, 0)),
            pl.BlockSpec((_BLK, 1), lambda i: (i, 0)),
            pl.BlockSpec((H, H), lambda i: (0, 0)),
            pl.BlockSpec((1, H), lambda i: (0, 0)),
            pl.BlockSpec((1, H), lambda i: (0, 0)),
            pl.BlockSpec((1, H), lambda i: (0, 0)),
            pl.BlockSpec((H, C), lambda i: (0, 0)),
            pl.BlockSpec((1, C), lambda i: (0, 0)),
        ],
        out_specs=pl.BlockSpec((G, C), lambda i: (0, 0)),
        out_shape=jax.ShapeDtypeStruct((G, C), jnp.float32),
        scratch_shapes=[
            pltpu.VMEM((G, H), jnp.float32),
            pltpu.VMEM((1, G), jnp.float32),
        ],
    )(h, batch2d, lw1, lb1.reshape(1, H), g4.reshape(1, H), be4.reshape(1, H),
      lw2, lb2.reshape(1, C))


def kernel(x, edge_index, batch, W1, b1, g1, be1, W2, b2, g2, be2,
           W3, b3, g3, be3, lw1, lb1, g4, be4, lw2, lb2):
    row = edge_index[0]
    col = edge_index[1]
    pad = E_PAD - E
    row3 = jnp.concatenate([row, jnp.zeros((pad,), jnp.int32)]).reshape(NW, CPT, K)
    col3 = jnp.concatenate([col, jnp.full((pad,), DUMP, jnp.int32)]).reshape(NW, CPT, K)
    x_p = jnp.concatenate([x, jnp.zeros((NP - N, x.shape[1]), x.dtype)])
    batch2d = jnp.concatenate([batch, jnp.full((NP - N,), G, batch.dtype)])
    batch2d = batch2d.reshape(NP, 1)
    zrows = jnp.zeros((ZB, H), jnp.float32)

    degs = _sc_degree(col3)
    dinv = _tc_dinv(degs)

    h = x_p
    for W, b, g, be in ((W1, b1, g1, be1), (W2, b2, g2, be2), (W3, b3, g3, be3)):
        xs = _tc_matmul_scale(h, W, dinv)
        p = _sc_aggregate(xs, row3, col3, zrows)
        h = _tc_combine(p, xs, dinv, b, g, be)

    return _tc_pool_head(h, batch2d, lw1, lb1, g4, be4, lw2, lb2)


# speedup vs baseline: 1.6523x; 1.0027x over previous
import dataclasses
import functools

import jax
import jax.numpy as jnp
from jax import lax
from jax.experimental import pallas as pl
from jax.experimental.pallas import tpu as pltpu
from jax.experimental.pallas import tpu_sc as plsc

N = 10000
E = 320000
H = 128
G = 64
C = 10
EPS = 1e-5

NC = 2
NS = 16
NW = NC * NS
K = 128
CPT = 79
E_PAD = NW * CPT * K
NP = 10240
RPT = NP // NS
ZB = 128
DUMP = NP - 1

_mesh = plsc.VectorSubcoreMesh(
    core_axis_name="c", subcore_axis_name="s", num_cores=NC, num_subcores=NS
)
_sc_no_layout = dataclasses.replace(
    pltpu.CompilerParams(), needs_layout_passes=False
)


@functools.partial(
    pl.kernel,
    out_type=jax.ShapeDtypeStruct((NW, NP), jnp.float32),
    mesh=_mesh,
    scratch_types=[
        pltpu.VMEM((K,), jnp.int32),
        pltpu.VMEM((NP,), jnp.float32),
    ],
    compiler_params=_sc_no_layout,
)
def _sc_degree(col_hbm, out, colbuf, hist_v):
    cid = lax.axis_index("c")
    sid = lax.axis_index("s")
    wid = cid * NS + sid

    @pl.loop(0, NP, step=16)
    def _(r):
        hist_v[pl.ds(r, 16)] = jnp.zeros((16,), jnp.float32)

    ones16 = jnp.ones((16,), jnp.float32)

    @pl.loop(0, CPT)
    def _(j):
        pltpu.sync_copy(col_hbm.at[wid, j], colbuf)

        @pl.loop(0, K, step=16)
        def _(q):
            idx = colbuf[pl.ds(q, 16)]
            plsc.addupdate_scatter(hist_v, [idx], ones16)

    pltpu.sync_copy(hist_v, out.at[wid])


@functools.partial(
    pl.kernel,
    out_type=jax.ShapeDtypeStruct((2 * NP, H), jnp.float32),
    mesh=_mesh,
    scratch_types=[
        pltpu.VMEM((K,), jnp.int32),
        pltpu.VMEM((K,), jnp.int32),
        pltpu.VMEM((K, H), jnp.float32),
        pltpu.VMEM((ZB, H), jnp.float32),
        pltpu.VMEM_SHARED((NP, H), jnp.float32),
        pltpu.SemaphoreType.DMA,
    ],
)
def _sc_aggregate(xs_hbm, row_hbm, col_hbm, zero_hbm, out, rowbuf, colbuf,
                  buf, z_v, acc, sem):
    cid = lax.axis_index("c")
    sid = lax.axis_index("s")
    wid = cid * NS + sid

    pltpu.sync_copy(zero_hbm, z_v)

    @pl.loop(0, RPT, step=ZB)
    def _(r):
        pltpu.sync_copy(z_v, acc.at[pl.ds(sid * RPT + r, ZB)])

    plsc.subcore_barrier()

    @pl.loop(0, CPT)
    def _(j):
        pltpu.sync_copy(row_hbm.at[wid, j], rowbuf)
        pltpu.sync_copy(col_hbm.at[wid, j], colbuf)
        pltpu.async_copy(xs_hbm.at[rowbuf], buf, sem).wait()
        pltpu.sync_copy(buf, acc.at[colbuf], add=True)

    plsc.subcore_barrier()

    @pl.loop(0, RPT, step=ZB)
    def _(r):
        pltpu.sync_copy(acc.at[pl.ds(sid * RPT + r, ZB)], z_v)
        pltpu.sync_copy(z_v, out.at[pl.ds(cid * NP + sid * RPT + r, ZB)])


_BLK = 1024
_NBLK = NP // _BLK


def _tc_dinv(degs):
    def body(d_ref, o_ref):
        s = jnp.sum(d_ref[...], axis=0, keepdims=True) + 1.0
        o_ref[...] = jnp.transpose(lax.rsqrt(s))

    return pl.pallas_call(
        body,
        out_shape=jax.ShapeDtypeStruct((NP, 1), jnp.float32),
    )(degs)


def _tc_matmul_scale(h, W, dinv):
    def body(h_ref, w_ref, d_ref, o_ref):
        o_ref[...] = d_ref[...] * jnp.dot(
            h_ref[...], w_ref[...], preferred_element_type=jnp.float32
        )

    return pl.pallas_call(
        body,
        grid=(_NBLK,),
        in_specs=[
            pl.BlockSpec((_BLK, H), lambda i: (i, 0)),
            pl.BlockSpec((H, H), lambda i: (0, 0)),
            pl.BlockSpec((_BLK, 1), lambda i: (i, 0)),
        ],
        out_specs=pl.BlockSpec((_BLK, H), lambda i: (i, 0)),
        out_shape=jax.ShapeDtypeStruct((NP, H), jnp.float32),
    )(h, W, dinv)


def _tc_combine(p, xs, dinv, b, g, be):
    def body(p0_ref, p1_ref, xs_ref, d_ref, b_ref, g_ref, be_ref, o_ref):
        gg = g_ref[...] * lax.rsqrt(jnp.float32(1.0 + EPS))
        agg = d_ref[...] * (p0_ref[...] + p1_ref[...] + xs_ref[...]) + b_ref[...]
        o_ref[...] = jnp.maximum(gg * agg + be_ref[...], 0.0)

    return pl.pallas_call(
        body,
        grid=(_NBLK,),
        in_specs=[
            pl.BlockSpec((_BLK, H), lambda i: (i, 0)),
            pl.BlockSpec((_BLK, H), lambda i: (_NBLK + i, 0)),
            pl.BlockSpec((_BLK, H), lambda i: (i, 0)),
            pl.BlockSpec((_BLK, 1), lambda i: (i, 0)),
            pl.BlockSpec((1, H), lambda i: (0, 0)),
            pl.BlockSpec((1, H), lambda i: (0, 0)),
            pl.BlockSpec((1, H), lambda i: (0, 0)),
        ],
        out_specs=pl.BlockSpec((_BLK, H), lambda i: (i, 0)),
        out_shape=jax.ShapeDtypeStruct((NP, H), jnp.float32),
    )(p, p, xs, dinv, b.reshape(1, H), g.reshape(1, H), be.reshape(1, H))


def _tc_pool_head(h, batch2d, lw1, lb1, g4, be4, lw2, lb2):
    def body(h_ref, b_ref, lw1_ref, lb1_ref, g4_ref, be4_ref, lw2_ref, lb2_ref,
             o_ref, sums, cnt):
        i = pl.program_id(0)

        @pl.when(i == 0)
        def _():
            sums[...] = jnp.zeros_like(sums)
            cnt[...] = jnp.zeros_like(cnt)

        seg = b_ref[...]
        onehot = (seg == lax.broadcasted_iota(jnp.int32, (_BLK, G), 1)).astype(
            jnp.float32
        )
        sums[...] += lax.dot_general(
            onehot, h_ref[...], (((0,), (0,)), ((), ())),
            preferred_element_type=jnp.float32,
        )
        cnt[...] += jnp.sum(onehot, axis=0, keepdims=True)

        @pl.when(i == _NBLK - 1)
        def _():
            p = sums[...] / jnp.maximum(cnt[...], 1.0).T
            gg = g4_ref[...] * lax.rsqrt(jnp.float32(1.0 + EPS))
            q = jnp.dot(p, lw1_ref[...], preferred_element_type=jnp.float32)
            q = jnp.maximum(gg * (q + lb1_ref[...]) + be4_ref[...], 0.0)
            o_ref[...] = (
                jnp.dot(q, lw2_ref[...], preferred_element_type=jnp.float32)
                + lb2_ref[...]
            )

    return pl.pallas_call(
        body,
        grid=(_NBLK,),
        in_specs=[
            pl.BlockSpec((_BLK, H), lambda i: (i, 0)),
            pl.BlockSpec((_BLK, 1), lambda i: (i, 0)),
            pl.BlockSpec((H, H), lambda i: (0, 0)),
            pl.BlockSpec((1, H), lambda i: (0, 0)),
            pl.BlockSpec((1, H), lambda i: (0, 0)),
            pl.BlockSpec((1, H), lambda i: (0, 0)),
            pl.BlockSpec((H, C), lambda i: (0, 0)),
            pl.BlockSpec((1, C), lambda i: (0, 0)),
        ],
        out_specs=pl.BlockSpec((G, C), lambda i: (0, 0)),
        out_shape=jax.ShapeDtypeStruct((G, C), jnp.float32),
        scratch_shapes=[
            pltpu.VMEM((G, H), jnp.float32),
            pltpu.VMEM((1, G), jnp.float32),
        ],
    )(h, batch2d, lw1, lb1.reshape(1, H), g4.reshape(1, H), be4.reshape(1, H),
      lw2, lb2.reshape(1, C))


def kernel(x, edge_index, batch, W1, b1, g1, be1, W2, b2, g2, be2,
           W3, b3, g3, be3, lw1, lb1, g4, be4, lw2, lb2):
    row = edge_index[0]
    col = edge_index[1]
    pad = E_PAD - E
    row3 = jnp.concatenate([row, jnp.zeros((pad,), jnp.int32)]).reshape(NW, CPT, K)
    col3 = jnp.concatenate([col, jnp.full((pad,), DUMP, jnp.int32)]).reshape(NW, CPT, K)
    x_p = jnp.concatenate([x, jnp.zeros((NP - N, x.shape[1]), x.dtype)])
    batch2d = jnp.concatenate([batch, jnp.full((NP - N,), G, batch.dtype)])
    batch2d = batch2d.reshape(NP, 1)
    zrows = jnp.zeros((ZB, H), jnp.float32)

    degs = _sc_degree(col3)
    dinv = _tc_dinv(degs)

    h = x_p
    for W, b, g, be in ((W1, b1, g1, be1), (W2, b2, g2, be2), (W3, b3, g3, be3)):
        xs = _tc_matmul_scale(h, W, dinv)
        p = _sc_aggregate(xs, row3, col3, zrows)
        h = _tc_combine(p, xs, dinv, b, g, be)

    return _tc_pool_head(h, batch2d, lw1, lb1, g4, be4, lw2, lb2)
